# baseline (speedup 1.0000x reference)
import functools

import jax
import jax.numpy as jnp
from jax import lax
from jax.experimental import pallas as pl
from jax.experimental.pallas import tpu as pltpu
from jax.experimental.pallas import tpu_sc as plsc

D = 128
S = 10
L = 16
NW = 32
C = 32
R = C * S
CHUNKS = 50
PER_TILE = C * CHUNKS
BPAD = PER_TILE * NW
GATHER_SPLITS = ((0, 128), (128, 128), (256, 64))


def _sc_mean(features, idx_flat):
    mesh = plsc.VectorSubcoreMesh(core_axis_name="c", subcore_axis_name="s")

    @functools.partial(
        pl.kernel,
        mesh=mesh,
        out_type=jax.ShapeDtypeStruct((BPAD, D), jnp.float32),
        scratch_types=[
            pltpu.VMEM((R,), jnp.int32),
            pltpu.VMEM((R,), jnp.int32),
            pltpu.VMEM((R, D), jnp.float32),
            pltpu.VMEM((R, D), jnp.float32),
            pltpu.VMEM((C, D), jnp.float32),
            pltpu.VMEM((C, D), jnp.float32),
            pltpu.SemaphoreType.DMA,
            pltpu.SemaphoreType.DMA,
            pltpu.SemaphoreType.DMA,
            pltpu.SemaphoreType.DMA,
            pltpu.SemaphoreType.DMA,
            pltpu.SemaphoreType.DMA,
        ],
    )
    def k(feat_hbm, idx_hbm, out_hbm, idx0, idx1, rows0, rows1, out0, out1,
          isem0, isem1, gsem0, gsem1, osem0, osem1):
        wid = lax.axis_index("s") * 2 + lax.axis_index("c")
        tile_node0 = wid * PER_TILE
        tile_row0 = tile_node0 * S
        idxs = (idx0, idx1)
        rows = (rows0, rows1)
        outs = (out0, out1)
        isems = (isem0, isem1)
        gsems = (gsem0, gsem1)
        osems = (osem0, osem1)

        def i_start(b, c):
            pltpu.async_copy(
                idx_hbm.at[pl.ds(tile_row0 + c * R, R)], idxs[b], isems[b])

        def i_wait(b, c):
            pltpu.make_async_copy(
                idx_hbm.at[pl.ds(tile_row0 + c * R, R)], idxs[b],
                isems[b]).wait()

        def g_start(b, c):
            for g0, gn in GATHER_SPLITS:
                pltpu.async_copy(
                    feat_hbm.at[idxs[b].at[pl.ds(g0, gn)]],
                    rows[b].at[pl.ds(g0, gn)],
                    gsems[b],
                )

        def g_wait(b, c):
            for g0, gn in GATHER_SPLITS:
                pltpu.make_async_copy(
                    feat_hbm.at[idxs[b].at[pl.ds(g0, gn)]],
                    rows[b].at[pl.ds(g0, gn)],
                    gsems[b],
                ).wait()

        def o_start(b, c):
            pltpu.async_copy(
                outs[b], out_hbm.at[pl.ds(tile_node0 + c * C, C)], osems[b])

        def o_wait(b, c):
            pltpu.make_async_copy(
                outs[b], out_hbm.at[pl.ds(tile_node0 + c * C, C)],
                osems[b]).wait()

        def compute(b):
            rows_b = rows[b]
            out_b = outs[b]

            def node_body(n, carry):
                base = n * S
                for col in range(D // L):
                    acc = rows_b[base, pl.ds(col * L, L)]
                    for s_ in range(1, S):
                        acc = acc + rows_b[base + s_, pl.ds(col * L, L)]
                    out_b[n, pl.ds(col * L, L)] = acc * jnp.float32(0.1)
                return carry

            lax.fori_loop(0, C, node_body, 0)

        pltpu.sync_copy(idx_hbm.at[pl.ds(tile_row0, R)], idx0)
        g_start(0, 0)
        i_start(1, 1)

        g_wait(0, 0)
        i_start(0, 2)
        i_wait(1, 1)
        g_start(1, 1)
        compute(0)
        o_start(0, 0)

        g_wait(1, 1)
        i_start(1, 3)
        i_wait(0, 2)
        g_start(0, 2)
        compute(1)
        o_start(1, 1)

        def steady(kk, carry):
            for b in (0, 1):
                c = 2 * kk + b
                g_wait(b, c)
                i_start(b, c + 2)
                i_wait(1 - b, c + 1)
                g_start(1 - b, c + 1)
                o_wait(b, c - 2)
                compute(b)
                o_start(b, c)
            return carry

        lax.fori_loop(1, CHUNKS // 2 - 1, steady, 0)

        g_wait(0, CHUNKS - 2)
        i_wait(1, CHUNKS - 1)
        g_start(1, CHUNKS - 1)
        o_wait(0, CHUNKS - 4)
        compute(0)
        o_start(0, CHUNKS - 2)

        g_wait(1, CHUNKS - 1)
        o_wait(1, CHUNKS - 3)
        compute(1)
        o_start(1, CHUNKS - 1)

        o_wait(0, CHUNKS - 2)
        o_wait(1, CHUNKS - 1)

    return k(features, idx_flat)


def kernel(features, nodes, to_neighs):
    b = to_neighs.shape[0]
    idx = to_neighs.astype(jnp.int32).reshape(-1)
    idx = jnp.pad(idx, (0, BPAD * S - idx.shape[0]))
    out = _sc_mean(features, idx)
    return out[:b]


# speedup vs baseline: 1.0024x; 1.0024x over previous
import functools

import jax
import jax.numpy as jnp
from jax import lax
from jax.experimental import pallas as pl
from jax.experimental.pallas import tpu as pltpu
from jax.experimental.pallas import tpu_sc as plsc

D = 128
S = 10
L = 16
NW = 32
C = 32
R = C * S
CHUNKS = 50
PER_TILE = C * CHUNKS
BPAD = PER_TILE * NW
GATHER_SPLITS = ((0, 128), (128, 128), (256, 64))


def _sc_mean(features, idx_flat):
    mesh = plsc.VectorSubcoreMesh(core_axis_name="c", subcore_axis_name="s")

    @functools.partial(
        pl.kernel,
        mesh=mesh,
        out_type=jax.ShapeDtypeStruct((BPAD, D), jnp.float32),
        scratch_types=[
            pltpu.VMEM((2 * R,), jnp.int32),
            pltpu.VMEM((2 * R, D), jnp.float32),
            pltpu.VMEM((2 * C, D), jnp.float32),
            pltpu.SemaphoreType.DMA,
            pltpu.SemaphoreType.DMA,
            pltpu.SemaphoreType.DMA,
        ],
    )
    def k(feat_hbm, idx_hbm, out_hbm, idx_v, rows_v, out_v, isem, gsem, osem):
        wid = lax.axis_index("s") * 2 + lax.axis_index("c")
        tile_node0 = wid * PER_TILE
        tile_row0 = tile_node0 * S

        def i_start(c, boff):
            pltpu.async_copy(
                idx_hbm.at[pl.ds(tile_row0 + c * R, R)],
                idx_v.at[pl.ds(boff, R)], isem)

        def i_wait():
            pltpu.make_async_copy(
                idx_hbm.at[pl.ds(tile_row0, R)],
                idx_v.at[pl.ds(0, R)], isem).wait()

        def g_start(boff):
            for g0, gn in GATHER_SPLITS:
                pltpu.async_copy(
                    feat_hbm.at[idx_v.at[pl.ds(boff + g0, gn)]],
                    rows_v.at[pl.ds(boff + g0, gn)],
                    gsem,
                )

        def g_wait(boff):
            for g0, gn in GATHER_SPLITS:
                pltpu.make_async_copy(
                    feat_hbm.at[idx_v.at[pl.ds(boff + g0, gn)]],
                    rows_v.at[pl.ds(boff + g0, gn)],
                    gsem,
                ).wait()

        def o_start(c, ooff):
            pltpu.async_copy(
                out_v.at[pl.ds(ooff, C)],
                out_hbm.at[pl.ds(tile_node0 + c * C, C)], osem)

        def o_wait():
            pltpu.make_async_copy(
                out_v.at[pl.ds(0, C)],
                out_hbm.at[pl.ds(tile_node0, C)], osem).wait()

        pltpu.sync_copy(idx_hbm.at[pl.ds(tile_row0, R)],
                        idx_v.at[pl.ds(0, R)])
        g_start(0)
        i_start(1, R)

        def chunk_body(c, carry):
            par = lax.rem(c, 2)
            boff = par * R
            boff_n = R - boff
            ooff = par * C

            g_wait(boff)
            @pl.when(c + 2 < CHUNKS)
            def _():
                i_start(c + 2, boff)

            @pl.when(c + 1 < CHUNKS)
            def _():
                i_wait()
                g_start(boff_n)

            @pl.when(c >= 2)
            def _():
                o_wait()

            def node_body(n, carry2):
                base = boff + n * S
                for col in range(D // L):
                    acc = rows_v[base, pl.ds(col * L, L)]
                    for s_ in range(1, S):
                        acc = acc + rows_v[base + s_, pl.ds(col * L, L)]
                    out_v[ooff + n, pl.ds(col * L, L)] = acc * jnp.float32(0.1)
                return carry2

            lax.fori_loop(0, C, node_body, 0)
            o_start(c, ooff)
            return carry

        lax.fori_loop(0, CHUNKS, chunk_body, 0)
        o_wait()
        o_wait()

    return k(features, idx_flat)


def kernel(features, nodes, to_neighs):
    b = to_neighs.shape[0]
    idx = to_neighs.astype(jnp.int32).reshape(-1)
    idx = jnp.pad(idx, (0, BPAD * S - idx.shape[0]))
    out = _sc_mean(features, idx)
    return out[:b]


# speedup vs baseline: 1.8757x; 1.8712x over previous
import functools

import jax
import jax.numpy as jnp
from jax import lax
from jax.experimental import pallas as pl
from jax.experimental.pallas import tpu as pltpu
from jax.experimental.pallas import tpu_sc as plsc

D = 128
S = 10
L = 16
NW = 32
C = 32
R = C * S
CHUNKS = 49
PER_TILE = C * CHUNKS
BPAD = PER_TILE * NW


def _sc_mean(features, idx_flat):
    mesh = plsc.VectorSubcoreMesh(core_axis_name="c", subcore_axis_name="s")

    @functools.partial(
        pl.kernel,
        mesh=mesh,
        out_type=jax.ShapeDtypeStruct((BPAD, D), jnp.float32),
        scratch_types=[
            pltpu.VMEM((R,), jnp.int32),
            pltpu.VMEM((R, D), jnp.float32),
            pltpu.VMEM((C, D), jnp.float32),
            pltpu.SemaphoreType.DMA,
        ],
    )
    def k(feat_hbm, idx_hbm, out_hbm, idx_v, rows_v, out_v, sem):
        wid = lax.axis_index("s") * 2 + lax.axis_index("c")
        tile_node0 = wid * PER_TILE

        def chunk_body(ci, carry):
            node0 = tile_node0 + ci * C
            row0 = node0 * S
            pltpu.sync_copy(idx_hbm.at[pl.ds(row0, R)], idx_v)
            cps = []
            for g0, gn in ((0, 128), (128, 128), (256, 64)):
                cps.append(pltpu.async_copy(
                    feat_hbm.at[idx_v.at[pl.ds(g0, gn)]],
                    rows_v.at[pl.ds(g0, gn)],
                    sem,
                ))
            for cp in cps:
                cp.wait()

            def node_body(n, carry2):
                base = n * S
                for c in range(D // L):
                    acc = rows_v[base, pl.ds(c * L, L)]
                    for s_ in range(1, S):
                        acc = acc + rows_v[base + s_, pl.ds(c * L, L)]
                    out_v[n, pl.ds(c * L, L)] = acc * jnp.float32(0.1)
                return carry2

            lax.fori_loop(0, C, node_body, 0)
            pltpu.sync_copy(out_v, out_hbm.at[pl.ds(node0, C)])
            return carry

        lax.fori_loop(0, CHUNKS, chunk_body, 0)

    return k(features, idx_flat)


def kernel(features, nodes, to_neighs):
    b = to_neighs.shape[0]
    idx = to_neighs.astype(jnp.int32).reshape(-1)
    idx = jnp.pad(idx, (0, BPAD * S - idx.shape[0]))
    out = _sc_mean(features, idx)
    return out[:b]
